# Initial kernel scaffold; baseline (speedup 1.0000x reference)
import functools

import jax
import jax.numpy as jnp
from jax import lax
from jax.experimental import pallas as pl
from jax.experimental.pallas import tpu as pltpu
from jax.experimental.pallas import tpu_sc as plsc

_N = 10000
_E = 320000
_D = 128
_H = 128
_G = 64

_NCORES = 2
_NSUB = 16
_NTILES = _NCORES * _NSUB
_CHUNK = 128
_CHUNKS_PER_TILE = 80
_EPAD = _NTILES * _CHUNKS_PER_TILE * _CHUNK
_NACC = 10016
_INIT_ROWS = _NACC // _NSUB
_OUT_ROWS = _N // _NSUB


def _agg_body(z_hbm, src_hbm, dst_hbm, zeros_hbm, out_hbm, sidx, didx, rows, acc, gsem):
    c = lax.axis_index("c")
    s = lax.axis_index("s")
    wid = s * _NCORES + c

    pltpu.sync_copy(zeros_hbm, acc.at[pl.ds(s * _INIT_ROWS, _INIT_ROWS)])

    pltpu.sync_copy(src_hbm.at[pl.ds(wid * _CHUNKS_PER_TILE, _CHUNKS_PER_TILE)], sidx)
    pltpu.sync_copy(dst_hbm.at[pl.ds(wid * _CHUNKS_PER_TILE, _CHUNKS_PER_TILE)], didx)

    plsc.subcore_barrier()

    def body(j, carry):
        pltpu.async_copy(z_hbm.at[sidx.at[j]], rows, gsem).wait()
        pltpu.sync_copy(rows, acc.at[didx.at[j]], add=True)
        return carry

    lax.fori_loop(0, _CHUNKS_PER_TILE, body, 0)

    plsc.subcore_barrier()

    pltpu.sync_copy(
        acc.at[pl.ds(s * _OUT_ROWS, _OUT_ROWS)],
        out_hbm.at[c, pl.ds(s * _OUT_ROWS, _OUT_ROWS)],
    )


_agg = functools.partial(
    pl.kernel,
    out_type=jax.ShapeDtypeStruct((_NCORES, _N, _D), jnp.float32),
    mesh=plsc.VectorSubcoreMesh(core_axis_name="c", subcore_axis_name="s"),
    scratch_types=[
        pltpu.VMEM((_CHUNKS_PER_TILE, _CHUNK), jnp.int32),
        pltpu.VMEM((_CHUNKS_PER_TILE, _CHUNK), jnp.int32),
        pltpu.VMEM((_CHUNK, _D), jnp.float32),
        pltpu.VMEM_SHARED((_NACC, _D), jnp.float32),
        pltpu.SemaphoreType.DMA,
    ],
)(_agg_body)


def _tc_layer_body(relu_out, z_ref, p_ref, w1_ref, b1_ref, w2_ref, b2_ref,
                   gm_ref, bt_ref, batch_ref, zout_ref, gout_ref):
    h = z_ref[...] + p_ref[0] + p_ref[1]
    h = jnp.dot(h, w1_ref[...], preferred_element_type=jnp.float32,
                precision=lax.Precision.HIGHEST) + b1_ref[...]
    h = jnp.maximum(h, 0.0)
    h = jnp.dot(h, w2_ref[...], preferred_element_type=jnp.float32,
                precision=lax.Precision.HIGHEST) + b2_ref[...]
    mean = jnp.mean(h, axis=0, keepdims=True)
    var = jnp.mean((h - mean) ** 2, axis=0, keepdims=True)
    h = (h - mean) * lax.rsqrt(var + 1e-5) * gm_ref[...] + bt_ref[...]
    if relu_out:
        h = jnp.maximum(h, 0.0)
    zout_ref[...] = h
    onehot = (lax.broadcasted_iota(jnp.int32, (_G, _N), 0) == batch_ref[...]
              ).astype(jnp.float32)
    gout_ref[...] = jnp.dot(onehot, h, preferred_element_type=jnp.float32,
                            precision=lax.Precision.HIGHEST)


_tc_layer = {
    flag: pl.pallas_call(
        functools.partial(_tc_layer_body, flag),
        out_shape=(
            jax.ShapeDtypeStruct((_N, _H), jnp.float32),
            jax.ShapeDtypeStruct((_G, _H), jnp.float32),
        ),
    )
    for flag in (False, True)
}


def kernel(x, edge_index, batch, W1_0, b1_0, W2_0, b2_0, gamma_0, beta_0,
           W1_1, b1_1, W2_1, b2_1, gamma_1, beta_1):
    src = edge_index[0]
    dst = edge_index[1]
    npad = _EPAD - _E
    src_p = jnp.concatenate([src, jnp.zeros((npad,), jnp.int32)]
                            ).reshape(_NTILES * _CHUNKS_PER_TILE, _CHUNK)
    dst_p = jnp.concatenate([dst, jnp.full((npad,), _N, jnp.int32)]
                            ).reshape(_NTILES * _CHUNKS_PER_TILE, _CHUNK)
    zeros = jnp.zeros((_INIT_ROWS, _D), jnp.float32)
    batch2 = batch.reshape(1, _N)

    params = [
        (W1_0, b1_0, W2_0, b2_0, gamma_0, beta_0),
        (W1_1, b1_1, W2_1, b2_1, gamma_1, beta_1),
    ]
    z = x
    zs, gs = [], []
    for i, (w1, b1, w2, b2, gm, bt) in enumerate(params):
        parts = _agg(z, src_p, dst_p, zeros)
        z, g = _tc_layer[i == 0](
            z, parts, w1, b1.reshape(1, _H), w2, b2.reshape(1, _H),
            gm.reshape(1, _H), bt.reshape(1, _H), batch2)
        zs.append(z)
        gs.append(g)
    return jnp.concatenate(zs, axis=1), jnp.concatenate(gs, axis=1)


# speedup vs baseline: 2.8303x; 2.8303x over previous
import functools

import jax
import jax.numpy as jnp
from jax import lax
from jax.experimental import pallas as pl
from jax.experimental.pallas import tpu as pltpu
from jax.experimental.pallas import tpu_sc as plsc

_N = 10000
_E = 320000
_D = 128
_H = 128
_G = 64

_NCORES = 2
_NSUB = 16
_NTILES = _NCORES * _NSUB
_CHUNK = 128
_CHUNKS_PER_TILE = 80
_EPAD = _NTILES * _CHUNKS_PER_TILE * _CHUNK
_NACC = 10240
_INIT_ROWS = _NACC // _NSUB
_OUT_ROWS = 624
_OUT_TAIL = _N - _NSUB * _OUT_ROWS


def _agg_body(z_hbm, src_hbm, dst_hbm, zeros_hbm, out_hbm, sidx, didx, rows, acc, gsem):
    c = lax.axis_index("c")
    s = lax.axis_index("s")
    wid = s * _NCORES + c

    pltpu.sync_copy(zeros_hbm, acc.at[pl.ds(s * _INIT_ROWS, _INIT_ROWS)])

    pltpu.sync_copy(src_hbm.at[pl.ds(wid * _CHUNKS_PER_TILE, _CHUNKS_PER_TILE)], sidx)
    pltpu.sync_copy(dst_hbm.at[pl.ds(wid * _CHUNKS_PER_TILE, _CHUNKS_PER_TILE)], didx)

    plsc.subcore_barrier()

    def body(j, carry):
        pltpu.async_copy(z_hbm.at[sidx.at[j]], rows, gsem).wait()
        pltpu.sync_copy(rows, acc.at[didx.at[j]], add=True)
        return carry

    lax.fori_loop(0, _CHUNKS_PER_TILE, body, 0)

    plsc.subcore_barrier()

    pltpu.sync_copy(
        acc.at[pl.ds(s * _OUT_ROWS, _OUT_ROWS)],
        out_hbm.at[c, pl.ds(s * _OUT_ROWS, _OUT_ROWS)],
    )

    @pl.when(s == _NSUB - 1)
    def _():
        pltpu.sync_copy(
            acc.at[pl.ds(_NSUB * _OUT_ROWS, _OUT_TAIL)],
            out_hbm.at[c, pl.ds(_NSUB * _OUT_ROWS, _OUT_TAIL)],
        )


@functools.cache
def _agg():
    return functools.partial(
        pl.kernel,
        out_type=jax.ShapeDtypeStruct((_NCORES, _N, _D), jnp.float32),
        mesh=plsc.VectorSubcoreMesh(core_axis_name="c", subcore_axis_name="s",
                                    num_cores=_NCORES, num_subcores=_NSUB),
        scratch_types=[
            pltpu.VMEM((_CHUNKS_PER_TILE, _CHUNK), jnp.int32),
            pltpu.VMEM((_CHUNKS_PER_TILE, _CHUNK), jnp.int32),
            pltpu.VMEM((_CHUNK, _D), jnp.float32),
            pltpu.VMEM_SHARED((_NACC, _D), jnp.float32),
            pltpu.SemaphoreType.DMA,
        ],
    )(_agg_body)


def _tc_layer_body(relu_out, z_ref, p_ref, w1_ref, b1_ref, w2_ref, b2_ref,
                   gm_ref, bt_ref, batch_ref, zout_ref, gout_ref):
    h = z_ref[...] + p_ref[0] + p_ref[1]
    h = jnp.dot(h, w1_ref[...], preferred_element_type=jnp.float32,
                precision=lax.Precision.HIGHEST) + b1_ref[...]
    h = jnp.maximum(h, 0.0)
    h = jnp.dot(h, w2_ref[...], preferred_element_type=jnp.float32,
                precision=lax.Precision.HIGHEST) + b2_ref[...]
    mean = jnp.mean(h, axis=0, keepdims=True)
    var = jnp.mean((h - mean) ** 2, axis=0, keepdims=True)
    h = (h - mean) * lax.rsqrt(var + 1e-5) * gm_ref[...] + bt_ref[...]
    if relu_out:
        h = jnp.maximum(h, 0.0)
    zout_ref[...] = h
    onehot = (lax.broadcasted_iota(jnp.int32, (_G, _N), 0) == batch_ref[...]
              ).astype(jnp.float32)
    gout_ref[...] = jnp.dot(onehot, h, preferred_element_type=jnp.float32,
                            precision=lax.Precision.HIGHEST)


_tc_layer = {
    flag: pl.pallas_call(
        functools.partial(_tc_layer_body, flag),
        out_shape=(
            jax.ShapeDtypeStruct((_N, _H), jnp.float32),
            jax.ShapeDtypeStruct((_G, _H), jnp.float32),
        ),
    )
    for flag in (False, True)
}


def kernel(x, edge_index, batch, W1_0, b1_0, W2_0, b2_0, gamma_0, beta_0,
           W1_1, b1_1, W2_1, b2_1, gamma_1, beta_1):
    src = edge_index[0]
    dst = edge_index[1]
    npad = _EPAD - _E
    src_p = jnp.concatenate([src, jnp.zeros((npad,), jnp.int32)]
                            ).reshape(_NTILES * _CHUNKS_PER_TILE, _CHUNK)
    dst_p = jnp.concatenate([dst, jnp.full((npad,), _N, jnp.int32)]
                            ).reshape(_NTILES * _CHUNKS_PER_TILE, _CHUNK)
    zeros = jnp.zeros((_INIT_ROWS, _D), jnp.float32)
    batch2 = batch.reshape(1, _N)

    params = [
        (W1_0, b1_0, W2_0, b2_0, gamma_0, beta_0),
        (W1_1, b1_1, W2_1, b2_1, gamma_1, beta_1),
    ]
    z = x
    zs, gs = [], []
    for i, (w1, b1, w2, b2, gm, bt) in enumerate(params):
        parts = _agg()(z, src_p, dst_p, zeros)
        z, g = _tc_layer[i == 0](
            z, parts, w1, b1.reshape(1, _H), w2, b2.reshape(1, _H),
            gm.reshape(1, _H), bt.reshape(1, _H), batch2)
        zs.append(z)
        gs.append(g)
    return jnp.concatenate(zs, axis=1), jnp.concatenate(gs, axis=1)


# speedup vs baseline: 5.5723x; 1.9688x over previous
import functools

import jax
import jax.numpy as jnp
from jax import lax
from jax.experimental import pallas as pl
from jax.experimental.pallas import tpu as pltpu
from jax.experimental.pallas import tpu_sc as plsc

_N = 10000
_E = 320000
_D = 128
_H = 128
_G = 64

_NCORES = 2
_NSUB = 16
_NTILES = _NCORES * _NSUB
_CHUNK = 96
_CHUNKS_PER_TILE = 105
_EPAD = _NTILES * _CHUNKS_PER_TILE * _CHUNK
_EPT = _CHUNKS_PER_TILE * _CHUNK
_NACC = 10112
_INIT_ROWS = _NACC // _NSUB
_OUT_ROWS = 624
_OUT_TAIL = _N - _NSUB * _OUT_ROWS


def _agg_body(z_hbm, src_hbm, dst_hbm, zeros_hbm, out_hbm, sidx, didx,
              rows, acc, gsem):
    c = lax.axis_index("c")
    s = lax.axis_index("s")
    wid = s * _NCORES + c

    pltpu.sync_copy(zeros_hbm, acc.at[pl.ds(s * _INIT_ROWS, _INIT_ROWS)])

    pltpu.sync_copy(src_hbm.at[pl.ds(wid * _EPT, _EPT)], sidx)
    pltpu.sync_copy(dst_hbm.at[pl.ds(wid * _EPT, _EPT)], didx)

    plsc.subcore_barrier()

    pltpu.async_copy(
        z_hbm.at[sidx.at[pl.ds(0, _CHUNK)]], rows.at[pl.ds(0, _CHUNK)], gsem)

    def body(j, carry):
        cur = pl.multiple_of(lax.rem(j, 2) * _CHUNK, _CHUNK)
        nxt = pl.multiple_of(lax.rem(j + 1, 2) * _CHUNK, _CHUNK)
        jn = jnp.minimum(j + 1, _CHUNKS_PER_TILE - 1)
        pltpu.make_async_copy(
            z_hbm.at[sidx.at[pl.ds(0, _CHUNK)]],
            rows.at[pl.ds(cur, _CHUNK)], gsem).wait()
        pltpu.async_copy(
            z_hbm.at[sidx.at[pl.ds(jn * _CHUNK, _CHUNK)]],
            rows.at[pl.ds(nxt, _CHUNK)], gsem)
        pltpu.sync_copy(
            rows.at[pl.ds(cur, _CHUNK)],
            acc.at[didx.at[pl.ds(j * _CHUNK, _CHUNK)]], add=True)
        return carry

    lax.fori_loop(0, _CHUNKS_PER_TILE, body, 0)

    pltpu.make_async_copy(
        z_hbm.at[sidx.at[pl.ds(0, _CHUNK)]],
        rows.at[pl.ds((_CHUNKS_PER_TILE % 2) * _CHUNK, _CHUNK)], gsem).wait()

    plsc.subcore_barrier()

    pltpu.sync_copy(
        acc.at[pl.ds(s * _OUT_ROWS, _OUT_ROWS)],
        out_hbm.at[c, pl.ds(s * _OUT_ROWS, _OUT_ROWS)],
    )

    @pl.when(s == _NSUB - 1)
    def _():
        pltpu.sync_copy(
            acc.at[pl.ds(_NSUB * _OUT_ROWS, _OUT_TAIL)],
            out_hbm.at[c, pl.ds(_NSUB * _OUT_ROWS, _OUT_TAIL)],
        )


@functools.cache
def _agg():
    return functools.partial(
        pl.kernel,
        out_type=jax.ShapeDtypeStruct((_NCORES, _N, _D), jnp.float32),
        mesh=plsc.VectorSubcoreMesh(core_axis_name="c", subcore_axis_name="s",
                                    num_cores=_NCORES, num_subcores=_NSUB),
        scratch_types=[
            pltpu.VMEM((_EPT,), jnp.int32),
            pltpu.VMEM((_EPT,), jnp.int32),
            pltpu.VMEM((2 * _CHUNK, _D), jnp.float32),
            pltpu.VMEM_SHARED((_NACC, _D), jnp.float32),
            pltpu.SemaphoreType.DMA,
        ],
    )(_agg_body)


def _tc_layer_body(relu_out, z_ref, p_ref, w1_ref, b1_ref, w2_ref, b2_ref,
                   gm_ref, bt_ref, batch_ref, zout_ref, gout_ref):
    h = z_ref[...] + p_ref[0] + p_ref[1]
    h = jnp.dot(h, w1_ref[...], preferred_element_type=jnp.float32) + b1_ref[...]
    h = jnp.maximum(h, 0.0)
    h = jnp.dot(h, w2_ref[...], preferred_element_type=jnp.float32) + b2_ref[...]
    mean = jnp.mean(h, axis=0, keepdims=True)
    var = jnp.mean((h - mean) ** 2, axis=0, keepdims=True)
    h = (h - mean) / jnp.sqrt(var + 1e-5) * gm_ref[...] + bt_ref[...]
    if relu_out:
        h = jnp.maximum(h, 0.0)
    zout_ref[...] = h
    onehot = (lax.broadcasted_iota(jnp.int32, (_G, _N), 0) == batch_ref[...]
              ).astype(jnp.float32)
    gout_ref[...] = jnp.dot(onehot, h, preferred_element_type=jnp.float32)


_tc_layer = {
    flag: pl.pallas_call(
        functools.partial(_tc_layer_body, flag),
        out_shape=(
            jax.ShapeDtypeStruct((_N, _H), jnp.float32),
            jax.ShapeDtypeStruct((_G, _H), jnp.float32),
        ),
    )
    for flag in (False, True)
}


def kernel(x, edge_index, batch, W1_0, b1_0, W2_0, b2_0, gamma_0, beta_0,
           W1_1, b1_1, W2_1, b2_1, gamma_1, beta_1):
    src = edge_index[0]
    dst = edge_index[1]
    npad = _EPAD - _E
    src_p = jnp.concatenate([src, jnp.zeros((npad,), jnp.int32)])
    dst_p = jnp.concatenate([dst, jnp.full((npad,), _N, jnp.int32)])
    zeros = jnp.zeros((_INIT_ROWS, _D), jnp.float32)
    batch2 = batch.reshape(1, _N)

    params = [
        (W1_0, b1_0, W2_0, b2_0, gamma_0, beta_0),
        (W1_1, b1_1, W2_1, b2_1, gamma_1, beta_1),
    ]
    z = x
    zs, gs = [], []
    for i, (w1, b1, w2, b2, gm, bt) in enumerate(params):
        parts = _agg()(z, src_p, dst_p, zeros)
        z, g = _tc_layer[i == 0](
            z, parts, w1, b1.reshape(1, _H), w2, b2.reshape(1, _H),
            gm.reshape(1, _H), bt.reshape(1, _H), batch2)
        zs.append(z)
        gs.append(g)
    return jnp.concatenate(zs, axis=1), jnp.concatenate(gs, axis=1)
